# baseline (speedup 1.0000x reference)
import functools

import jax
import jax.numpy as jnp
from jax import lax
from jax.experimental import pallas as pl
from jax.experimental.pallas import tpu as pltpu
from jax.experimental.pallas import tpu_sc as plsc

NC = 2
NS = 16
NW = NC * NS
CH = 128
NBUF = 4
DBLK = 4000
SBLK = 512

_MESH = plsc.VectorSubcoreMesh(
    core_axis_name="c", subcore_axis_name="s", num_cores=NC, num_subcores=NS
)


def _detile_tc(weight):
    n, d = weight.shape
    grid = n // DBLK

    def body(a_ref, o_ref):
        o_ref[...] = jnp.concatenate([a_ref[...], a_ref[...]], axis=1)

    return pl.pallas_call(
        body,
        grid=(grid,),
        in_specs=[pl.BlockSpec((DBLK, d), lambda i: (i, 0))],
        out_specs=pl.BlockSpec((DBLK, 2 * d), lambda i: (i, 0)),
        out_shape=jax.ShapeDtypeStruct((n, 2 * d), jnp.float32),
    )(weight)


def _select_relayout_tc(pairs, shape):
    ns, nf, d = shape
    rpb = SBLK * nf

    def body(i_ref, o_ref):
        x = i_ref[...].reshape(SBLK, nf, 2 * d)
        o_ref[...] = x[:, :, :d]

    return pl.pallas_call(
        body,
        grid=(ns // SBLK,),
        in_specs=[pl.BlockSpec((rpb, 128), lambda i: (i, 0))],
        out_specs=pl.BlockSpec((SBLK, nf, d), lambda i: (i, 0, 0)),
        out_shape=jax.ShapeDtypeStruct(shape, jnp.float32),
    )(pairs)


def _flat_gather(t2, idxp, B, cpw):
    rounds = cpw // NBUF

    @functools.partial(
        pl.kernel,
        mesh=_MESH,
        out_type=jax.ShapeDtypeStruct((B, 128), jnp.float32),
        scratch_types=[
            pltpu.VMEM((cpw, CH), jnp.int32),
            pltpu.VMEM((NBUF, CH, 128), jnp.float32),
            pltpu.SemaphoreType.DMA((NBUF,)),
        ],
        compiler_params=pltpu.CompilerParams(use_tc_tiling_on_sc=False),
    )
    def k(t2_hbm, idx_hbm, out_hbm, idx_v, bufs, gsem):
        wid = lax.axis_index("s") * NC + lax.axis_index("c")
        pltpu.sync_copy(idx_hbm.at[wid], idx_v)
        base = wid * cpw

        for b in range(NBUF):
            pltpu.make_async_copy(
                t2_hbm.at[idx_v.at[b]], bufs.at[b], gsem.at[b]
            ).start()

        def round_body(r, carry):
            for b in range(NBUF):
                j = r * NBUF + b
                pltpu.make_async_copy(
                    t2_hbm.at[idx_v.at[j]], bufs.at[b], gsem.at[b]
                ).wait()
                pltpu.sync_copy(
                    bufs.at[b], out_hbm.at[pl.ds((base + j) * CH, CH)]
                )
                pltpu.make_async_copy(
                    t2_hbm.at[idx_v.at[j + NBUF]], bufs.at[b], gsem.at[b]
                ).start()
            return carry

        lax.fori_loop(0, rounds - 1, round_body, 0)

        for b in range(NBUF):
            j = (rounds - 1) * NBUF + b
            pltpu.make_async_copy(
                t2_hbm.at[idx_v.at[j]], bufs.at[b], gsem.at[b]
            ).wait()
            pltpu.sync_copy(
                bufs.at[b], out_hbm.at[pl.ds((base + j) * CH, CH)]
            )

    return k(t2, idxp)


def kernel(indices, weight):
    B = indices.size
    d = weight.shape[1]
    cpw = B // (NW * CH)
    idxp = indices.reshape(NW, cpw, CH).astype(jnp.int32)
    t2 = _detile_tc(weight)
    pairs = _flat_gather(t2, idxp, B, cpw)
    return _select_relayout_tc(pairs, indices.shape + (d,))


# speedup vs baseline: 1.0991x; 1.0991x over previous
import functools

import jax
import jax.numpy as jnp
from jax import lax
from jax.experimental import pallas as pl
from jax.experimental.pallas import tpu as pltpu
from jax.experimental.pallas import tpu_sc as plsc

NC = 2
NS = 16
NW = NC * NS
CH = 128
NBUF = 4
DBLK = 4000
SBLK = 512

_MESH = plsc.VectorSubcoreMesh(
    core_axis_name="c", subcore_axis_name="s", num_cores=NC, num_subcores=NS
)


def _select_relayout_tc(pairs, parity, shape):
    ns, nf, d = shape
    rpb = SBLK * nf

    def body(i_ref, p_ref, o_ref):
        x = i_ref[...].reshape(SBLK, nf, 2 * d)
        p = p_ref[...][:, :, None] != 0
        o_ref[...] = jnp.where(p, x[:, :, d:], x[:, :, :d])

    return pl.pallas_call(
        body,
        grid=(ns // SBLK,),
        in_specs=[
            pl.BlockSpec((rpb, 128), lambda i: (i, 0)),
            pl.BlockSpec((SBLK, nf), lambda i: (i, 0)),
        ],
        out_specs=pl.BlockSpec((SBLK, nf, d), lambda i: (i, 0, 0)),
        out_shape=jax.ShapeDtypeStruct(shape, jnp.float32),
    )(pairs, parity)


def _flat_gather(t2, idxp, B, cpw):
    rounds = cpw // NBUF

    @functools.partial(
        pl.kernel,
        mesh=_MESH,
        out_type=jax.ShapeDtypeStruct((B, 128), jnp.float32),
        scratch_types=[
            pltpu.VMEM((cpw, CH), jnp.int32),
            pltpu.VMEM((NBUF, CH, 128), jnp.float32),
            pltpu.SemaphoreType.DMA((NBUF,)),
        ],
        compiler_params=pltpu.CompilerParams(use_tc_tiling_on_sc=False),
    )
    def k(t2_hbm, idx_hbm, out_hbm, idx_v, bufs, gsem):
        wid = lax.axis_index("s") * NC + lax.axis_index("c")
        pltpu.sync_copy(idx_hbm.at[wid], idx_v)
        base = wid * cpw

        for b in range(NBUF):
            pltpu.make_async_copy(
                t2_hbm.at[idx_v.at[b]], bufs.at[b], gsem.at[b]
            ).start()

        def round_body(r, carry):
            for b in range(NBUF):
                j = r * NBUF + b
                pltpu.make_async_copy(
                    t2_hbm.at[idx_v.at[j]], bufs.at[b], gsem.at[b]
                ).wait()
                pltpu.sync_copy(
                    bufs.at[b], out_hbm.at[pl.ds((base + j) * CH, CH)]
                )
                pltpu.make_async_copy(
                    t2_hbm.at[idx_v.at[j + NBUF]], bufs.at[b], gsem.at[b]
                ).start()
            return carry

        lax.fori_loop(0, rounds - 1, round_body, 0)

        for b in range(NBUF):
            j = (rounds - 1) * NBUF + b
            pltpu.make_async_copy(
                t2_hbm.at[idx_v.at[j]], bufs.at[b], gsem.at[b]
            ).wait()
            pltpu.sync_copy(
                bufs.at[b], out_hbm.at[pl.ds((base + j) * CH, CH)]
            )

    return k(t2, idxp)


def kernel(indices, weight):
    B = indices.size
    d = weight.shape[1]
    cpw = B // (NW * CH)
    flat = indices.reshape(-1).astype(jnp.int32)
    idxp = (flat // 2).reshape(NW, cpw, CH)
    parity = (indices % 2).astype(jnp.int32)
    t2 = weight.reshape(weight.shape[0] // 2, 2 * d)
    pairs = _flat_gather(t2, idxp, B, cpw)
    return _select_relayout_tc(pairs, parity, indices.shape + (d,))


# speedup vs baseline: 1.2193x; 1.1094x over previous
import functools

import jax
import jax.numpy as jnp
from jax import lax
from jax.experimental import pallas as pl
from jax.experimental.pallas import tpu as pltpu
from jax.experimental.pallas import tpu_sc as plsc

NC = 2
NS = 16
NW = NC * NS
CH = 128
NBUF = 4

_MESH = plsc.VectorSubcoreMesh(
    core_axis_name="c", subcore_axis_name="s", num_cores=NC, num_subcores=NS
)


def _flat_gather(tpad, idx3, B, d, cpw):
    rounds = cpw // NBUF

    @functools.partial(
        pl.kernel,
        mesh=_MESH,
        out_type=jax.ShapeDtypeStruct((B, 128), jnp.float32),
        scratch_types=[
            pltpu.VMEM((cpw, CH), jnp.int32),
            pltpu.VMEM((NBUF, CH, 128), jnp.float32),
            pltpu.SemaphoreType.DMA((NBUF,)),
        ],
        compiler_params=pltpu.CompilerParams(use_tc_tiling_on_sc=False),
    )
    def k(tpad_hbm, idx_hbm, out_hbm, idx_v, bufs, gsem):
        wid = lax.axis_index("s") * NC + lax.axis_index("c")
        pltpu.sync_copy(idx_hbm.at[wid], idx_v)
        base = wid * cpw

        for b in range(NBUF):
            pltpu.make_async_copy(
                tpad_hbm.at[idx_v.at[b]], bufs.at[b], gsem.at[b]
            ).start()

        def round_body(r, carry):
            for b in range(NBUF):
                j = r * NBUF + b
                pltpu.make_async_copy(
                    tpad_hbm.at[idx_v.at[j]], bufs.at[b], gsem.at[b]
                ).wait()
                pltpu.sync_copy(
                    bufs.at[b], out_hbm.at[pl.ds((base + j) * CH, CH)]
                )
                pltpu.make_async_copy(
                    tpad_hbm.at[idx_v.at[j + NBUF]], bufs.at[b], gsem.at[b]
                ).start()
            return carry

        lax.fori_loop(0, rounds - 1, round_body, 0)

        for b in range(NBUF):
            j = (rounds - 1) * NBUF + b
            pltpu.make_async_copy(
                tpad_hbm.at[idx_v.at[j]], bufs.at[b], gsem.at[b]
            ).wait()
            pltpu.sync_copy(
                bufs.at[b], out_hbm.at[pl.ds((base + j) * CH, CH)]
            )

    return k(tpad, idx3)


def kernel(indices, weight):
    B = indices.size
    n, d = weight.shape
    cpw = B // (NW * CH)
    idx3 = indices.reshape(NW, cpw, CH).astype(jnp.int32)
    tpad = jnp.pad(weight, ((0, 0), (0, 128 - d)))
    out = _flat_gather(tpad, idx3, B, d, cpw)
    return out[:, :d].reshape(indices.shape + (d,))


# speedup vs baseline: 1.3007x; 1.0667x over previous
import functools

import jax
import jax.numpy as jnp
from jax import lax
from jax.experimental import pallas as pl
from jax.experimental.pallas import tpu as pltpu
from jax.experimental.pallas import tpu_sc as plsc

NC = 2
NS = 16
NW = NC * NS
CH = 128
NBUF = 4


def _flat_gather(weight, idx3, B, D, cpw):
    rounds = cpw // NBUF
    mesh = plsc.VectorSubcoreMesh(
        core_axis_name="c", subcore_axis_name="s", num_cores=NC, num_subcores=NS
    )

    @functools.partial(
        pl.kernel,
        mesh=mesh,
        out_type=jax.ShapeDtypeStruct((B, D), jnp.float32),
        scratch_types=[
            pltpu.VMEM((cpw, CH), jnp.int32),
            pltpu.VMEM((NBUF, CH, D), jnp.float32),
            pltpu.SemaphoreType.DMA((NBUF,)),
        ],
        compiler_params=pltpu.CompilerParams(use_tc_tiling_on_sc=False),
    )
    def k(table_hbm, idx_hbm, out_hbm, idx_v, bufs, gsem):
        wid = lax.axis_index("s") * NC + lax.axis_index("c")
        pltpu.sync_copy(idx_hbm.at[wid], idx_v)
        base = wid * cpw

        for b in range(NBUF):
            pltpu.make_async_copy(
                table_hbm.at[idx_v.at[b]], bufs.at[b], gsem.at[b]
            ).start()

        def round_body(r, carry):
            for b in range(NBUF):
                j = r * NBUF + b
                pltpu.make_async_copy(
                    table_hbm.at[idx_v.at[j]], bufs.at[b], gsem.at[b]
                ).wait()
                pltpu.sync_copy(bufs.at[b], out_hbm.at[pl.ds((base + j) * CH, CH)])
                pltpu.make_async_copy(
                    table_hbm.at[idx_v.at[j + NBUF]], bufs.at[b], gsem.at[b]
                ).start()
            return carry

        lax.fori_loop(0, rounds - 1, round_body, 0)

        for b in range(NBUF):
            j = (rounds - 1) * NBUF + b
            pltpu.make_async_copy(
                table_hbm.at[idx_v.at[j]], bufs.at[b], gsem.at[b]
            ).wait()
            pltpu.sync_copy(bufs.at[b], out_hbm.at[pl.ds((base + j) * CH, CH)])

    return k(weight, idx3)


def kernel(indices, weight):
    B = indices.size
    D = weight.shape[1]
    cpw = B // (NW * CH)
    idx3 = indices.reshape(NW, cpw, CH).astype(jnp.int32)
    out = _flat_gather(weight, idx3, B, D, cpw)
    return out.reshape(indices.shape + (D,))
